# baseline (speedup 1.0000x reference)
import functools

import jax
import jax.numpy as jnp
from jax import lax
from jax.experimental import pallas as pl
from jax.experimental.pallas import tpu as pltpu
from jax.experimental.pallas import tpu_sc as plsc


def _sc_gather(dt_table, rd_table, idx_dt, idx_rd):
    B = idx_dt.shape[0]
    d_dt = dt_table.shape[1]
    d_rd = rd_table.shape[1]
    info = plsc.get_sparse_core_info()
    nw = info.num_cores * info.num_subcores
    bpw = B // nw

    mesh = plsc.VectorSubcoreMesh(core_axis_name="c", subcore_axis_name="s")

    @functools.partial(
        pl.kernel,
        mesh=mesh,
        out_type=(
            jax.ShapeDtypeStruct((B, d_dt), jnp.float32),
            jax.ShapeDtypeStruct((B, d_rd), jnp.float32),
        ),
        scratch_types=[
            pltpu.VMEM((bpw,), jnp.int32),
            pltpu.VMEM((bpw,), jnp.int32),
            pltpu.SemaphoreType.DMA,
            pltpu.SemaphoreType.DMA,
        ],
    )
    def gather_k(dt_hbm, rd_hbm, idt_hbm, ird_hbm, out_dt, out_rd,
                 idt_v, ird_v, sem_dt, sem_rd):
        wid = lax.axis_index("s") * info.num_cores + lax.axis_index("c")
        base = wid * bpw
        pltpu.sync_copy(idt_hbm.at[pl.ds(base, bpw)], idt_v)
        pltpu.sync_copy(ird_hbm.at[pl.ds(base, bpw)], ird_v)

        nl = info.num_lanes

        def fire(j, _):
            vi_dt = idt_v[pl.ds(j * nl, nl)]
            vi_rd = ird_v[pl.ds(j * nl, nl)]
            for k in range(nl):
                i = j * nl + k
                pltpu.async_copy(dt_hbm.at[pl.ds(vi_dt[k], 1), :],
                                 out_dt.at[pl.ds(base + i, 1), :], sem_dt)
                pltpu.async_copy(rd_hbm.at[pl.ds(vi_rd[k], 1), :],
                                 out_rd.at[pl.ds(base + i, 1), :], sem_rd)
            return _

        lax.fori_loop(0, bpw // nl, fire, None)
        pltpu.make_async_copy(dt_hbm.at[pl.ds(0, bpw), :],
                              out_dt.at[pl.ds(base, bpw), :], sem_dt).wait()
        pltpu.make_async_copy(rd_hbm.at[pl.ds(0, bpw), :],
                              out_rd.at[pl.ds(base, bpw), :], sem_rd).wait()

    return gather_k(dt_table, rd_table, idx_dt, idx_rd)


def _tc_mlp(xdt, xrd, w1, w2, b2d):
    B = xdt.shape[0]
    hid = w1.shape[1]
    blk = 2048
    grid = (B // blk,)

    def body(xdt_ref, xrd_ref, w1_ref, w2_ref, b_ref, o_ref):
        acc = jnp.dot(xdt_ref[...], w1_ref[...],
                      preferred_element_type=jnp.float32)
        acc += jnp.dot(xrd_ref[...], w2_ref[...],
                       preferred_element_type=jnp.float32)
        o_ref[...] = jnp.maximum(acc + b_ref[...], 0.0)

    return pl.pallas_call(
        body,
        grid=grid,
        in_specs=[
            pl.BlockSpec((blk, xdt.shape[1]), lambda i: (i, 0)),
            pl.BlockSpec((blk, xrd.shape[1]), lambda i: (i, 0)),
            pl.BlockSpec(w1.shape, lambda i: (0, 0)),
            pl.BlockSpec(w2.shape, lambda i: (0, 0)),
            pl.BlockSpec(b2d.shape, lambda i: (0, 0)),
        ],
        out_specs=pl.BlockSpec((blk, hid), lambda i: (i, 0)),
        out_shape=jax.ShapeDtypeStruct((B, hid), jnp.float32),
    )(xdt, xrd, w1, w2, b2d)


def kernel(x, dt_table, rd_table, W, b):
    d_dt = dt_table.shape[1]
    idx_dt = x[:, 0]
    idx_rd = x[:, 1]
    g_dt, g_rd = _sc_gather(dt_table, rd_table, idx_dt, idx_rd)
    w1 = W[:d_dt]
    w2 = W[d_dt:]
    return _tc_mlp(g_dt, g_rd, w1, w2, b.reshape(1, -1))


# speedup vs baseline: 2.2973x; 2.2973x over previous
import functools

import jax
import jax.numpy as jnp
from jax import lax
from jax.experimental import pallas as pl
from jax.experimental.pallas import tpu as pltpu
from jax.experimental.pallas import tpu_sc as plsc


def _sc_gather(dt_table, rd_table, idx_dt, idx_rd):
    B = idx_dt.shape[0]
    d_dt = dt_table.shape[1]
    d_rd = rd_table.shape[1]
    info = plsc.get_sparse_core_info()
    nw = info.num_cores * info.num_subcores
    nl = info.num_lanes
    bpw = B // nw
    chunk = bpw // 2

    mesh = plsc.VectorSubcoreMesh(core_axis_name="c", subcore_axis_name="s")

    @functools.partial(
        pl.kernel,
        mesh=mesh,
        out_type=(
            jax.ShapeDtypeStruct((B, d_dt), jnp.float32),
            jax.ShapeDtypeStruct((B, d_rd), jnp.float32),
        ),
        scratch_types=[
            pltpu.VMEM((bpw,), jnp.int32),
            pltpu.VMEM((bpw,), jnp.int32),
            pltpu.VMEM((chunk, d_dt), jnp.float32),
            pltpu.VMEM((chunk, d_rd), jnp.float32),
            pltpu.SemaphoreType.DMA,
            pltpu.SemaphoreType.DMA,
        ],
    )
    def gather_k(dt_hbm, rd_hbm, idt_hbm, ird_hbm, out_dt, out_rd,
                 idt_v, ird_v, dt_buf, rd_buf, sem_dt, sem_rd):
        wid = lax.axis_index("s") * info.num_cores + lax.axis_index("c")
        base = wid * bpw
        pltpu.sync_copy(idt_hbm.at[pl.ds(base, bpw)], idt_v)
        pltpu.sync_copy(ird_hbm.at[pl.ds(base, bpw)], ird_v)

        for half in range(2):
            def fire(j, _):
                vi_dt = idt_v[pl.ds(half * chunk + j * nl, nl)]
                vi_rd = ird_v[pl.ds(half * chunk + j * nl, nl)]
                for k in range(nl):
                    i = j * nl + k
                    pltpu.async_copy(dt_hbm.at[pl.ds(vi_dt[k], 1), :],
                                     dt_buf.at[pl.ds(i, 1), :], sem_dt)
                    pltpu.async_copy(rd_hbm.at[pl.ds(vi_rd[k], 1), :],
                                     rd_buf.at[pl.ds(i, 1), :], sem_rd)
                return _

            lax.fori_loop(0, chunk // nl, fire, None)
            pltpu.make_async_copy(dt_hbm.at[pl.ds(0, chunk), :], dt_buf,
                                  sem_dt).wait()
            pltpu.make_async_copy(rd_hbm.at[pl.ds(0, chunk), :], rd_buf,
                                  sem_rd).wait()
            off = base + half * chunk
            pltpu.sync_copy(dt_buf, out_dt.at[pl.ds(off, chunk)])
            pltpu.sync_copy(rd_buf, out_rd.at[pl.ds(off, chunk)])

    return gather_k(dt_table, rd_table, idx_dt, idx_rd)


def _tc_mlp(xdt, xrd, w1, w2, b2d):
    B = xdt.shape[0]
    hid = w1.shape[1]
    blk = 2048
    grid = (B // blk,)

    def body(xdt_ref, xrd_ref, w1_ref, w2_ref, b_ref, o_ref):
        acc = jnp.dot(xdt_ref[...], w1_ref[...],
                      preferred_element_type=jnp.float32)
        acc += jnp.dot(xrd_ref[...], w2_ref[...],
                       preferred_element_type=jnp.float32)
        o_ref[...] = jnp.maximum(acc + b_ref[...], 0.0)

    return pl.pallas_call(
        body,
        grid=grid,
        in_specs=[
            pl.BlockSpec((blk, xdt.shape[1]), lambda i: (i, 0)),
            pl.BlockSpec((blk, xrd.shape[1]), lambda i: (i, 0)),
            pl.BlockSpec(w1.shape, lambda i: (0, 0)),
            pl.BlockSpec(w2.shape, lambda i: (0, 0)),
            pl.BlockSpec(b2d.shape, lambda i: (0, 0)),
        ],
        out_specs=pl.BlockSpec((blk, hid), lambda i: (i, 0)),
        out_shape=jax.ShapeDtypeStruct((B, hid), jnp.float32),
    )(xdt, xrd, w1, w2, b2d)


def kernel(x, dt_table, rd_table, W, b):
    d_dt = dt_table.shape[1]
    idx_dt = x[:, 0]
    idx_rd = x[:, 1]
    g_dt, g_rd = _sc_gather(dt_table, rd_table, idx_dt, idx_rd)
    w1 = W[:d_dt]
    w2 = W[d_dt:]
    return _tc_mlp(g_dt, g_rd, w1, w2, b.reshape(1, -1))
